# baseline (speedup 1.0000x reference)
import jax
import jax.numpy as jnp
from jax.experimental import pallas as pl


def _body(gu_ref, gi_ref, xui_ref, u_ref, i_ref):
    u = gu_ref[...]
    v = gi_ref[...]
    u_ref[...] = u
    i_ref[...] = v
    ones = jnp.ones((u.shape[1],), dtype=u.dtype)
    xui_ref[...] = jax.lax.dot_general(
        u * v, ones, (((1,), (0,)), ((), ())),
        preferred_element_type=jnp.float32)


def kernel(gu, gi):
    B, D = gu.shape
    BS = 512
    xui, gamma_u, gamma_i = pl.pallas_call(
        _body,
        grid=(B // BS,),
        in_specs=[
            pl.BlockSpec((BS, D), lambda b: (b, 0)),
            pl.BlockSpec((BS, D), lambda b: (b, 0)),
        ],
        out_specs=[
            pl.BlockSpec((BS,), lambda b: (b,)),
            pl.BlockSpec((BS, D), lambda b: (b, 0)),
            pl.BlockSpec((BS, D), lambda b: (b, 0)),
        ],
        out_shape=[
            jax.ShapeDtypeStruct((B,), gu.dtype),
            jax.ShapeDtypeStruct((B, D), gu.dtype),
            jax.ShapeDtypeStruct((B, D), gi.dtype),
        ],
    )(gu, gi)
    return (xui, gamma_u, gamma_i)


# speedup vs baseline: 1.4440x; 1.4440x over previous
import jax
import jax.numpy as jnp
from jax.experimental import pallas as pl


def _body(gu_ref, gi_ref, xui_ref):
    u = gu_ref[...]
    v = gi_ref[...]
    ones = jnp.ones((u.shape[1],), dtype=u.dtype)
    xui_ref[...] = jax.lax.dot_general(
        u * v, ones, (((1,), (0,)), ((), ())),
        preferred_element_type=jnp.float32)


def kernel(gu, gi):
    B, D = gu.shape
    BS = 2048
    xui = pl.pallas_call(
        _body,
        grid=(B // BS,),
        in_specs=[
            pl.BlockSpec((BS, D), lambda b: (b, 0)),
            pl.BlockSpec((BS, D), lambda b: (b, 0)),
        ],
        out_specs=pl.BlockSpec((BS,), lambda b: (b,)),
        out_shape=jax.ShapeDtypeStruct((B,), gu.dtype),
    )(gu, gi)
    return (xui, gu, gi)
